# baseline (speedup 1.0000x reference)
import functools
import math

import jax
import jax.numpy as jnp
from jax import lax
from jax.experimental import pallas as pl
from jax.experimental.pallas import tpu as pltpu
from jax.experimental.pallas import tpu_sc as plsc

_EMB = 64
_B = 4096
_L = 200
_N = _B * _L
_NC = 2
_NS = 16
_NW = _NC * _NS
_PER_W = _N // _NW
_GSZ = 128
_G = 8
_C = _GSZ * _G
_NCHUNK = _PER_W // _C
_SCALE = math.sqrt(_EMB)
_LANES = 16


def _build():
    mesh = plsc.VectorSubcoreMesh(core_axis_name="c", subcore_axis_name="s")

    @functools.partial(
        pl.kernel,
        mesh=mesh,
        compiler_params=pltpu.CompilerParams(use_tc_tiling_on_sc=False),
        out_type=jax.ShapeDtypeStruct((_N, _EMB), jnp.float32),
        scratch_types=[
            pltpu.VMEM((_G, _GSZ), jnp.int32),
            pltpu.VMEM((_C, _EMB), jnp.float32),
            pltpu.SemaphoreType.DMA,
        ],
    )
    def emb(tok_hbm, table_hbm, out_hbm, idx_v, rows_v, gsem):
        wid = lax.axis_index("s") * _NC + lax.axis_index("c")
        base = wid * _PER_W

        def chunk_body(i, carry):
            off = pl.multiple_of(base + i * _C, _C)
            goff = pl.multiple_of(off // _GSZ, _G)
            pltpu.sync_copy(tok_hbm.at[pl.ds(goff, _G)], idx_v)
            handles = []
            for j in range(_G):
                handles.append(
                    pltpu.async_copy(
                        table_hbm.at[idx_v.at[j]],
                        rows_v.at[pl.ds(j * _GSZ, _GSZ)],
                        gsem,
                    )
                )
            for h in handles:
                h.wait()

            def row_body(r, c):
                for j in range(_EMB // _LANES):
                    sl = pl.ds(j * _LANES, _LANES)
                    rows_v[r, sl] = rows_v[r, sl] * _SCALE
                return c

            lax.fori_loop(0, _C, row_body, 0, unroll=4)
            pltpu.sync_copy(rows_v, out_hbm.at[pl.ds(off, _C)])
            return carry

        lax.fori_loop(0, _NCHUNK, chunk_body, 0)

    return emb


_emb = _build()


@jax.jit
def kernel(tokens, table):
    tok = tokens.reshape(_N // _GSZ, _GSZ).astype(jnp.int32)
    out = _emb(tok, table)
    return out.reshape(_B, _L, _EMB)


# speedup vs baseline: 1.0615x; 1.0615x over previous
import functools
import math

import jax
import jax.numpy as jnp
from jax import lax
from jax.experimental import pallas as pl
from jax.experimental.pallas import tpu as pltpu
from jax.experimental.pallas import tpu_sc as plsc

_EMB = 64
_B = 4096
_L = 200
_N = _B * _L
_NC = 2
_NS = 16
_NW = _NC * _NS
_PER_W = _N // _NW
_GSZ = 128
_G = 4
_C = _GSZ * _G
_NCHUNK = _PER_W // _C
_NPAIR = _NCHUNK // 2
_SCALE = math.sqrt(_EMB)
_LANES = 16


def _build():
    mesh = plsc.VectorSubcoreMesh(core_axis_name="c", subcore_axis_name="s")

    @functools.partial(
        pl.kernel,
        mesh=mesh,
        compiler_params=pltpu.CompilerParams(use_tc_tiling_on_sc=False),
        out_type=jax.ShapeDtypeStruct((_N, _EMB), jnp.float32),
        scratch_types=[
            pltpu.VMEM((_PER_W,), jnp.int32),
            pltpu.VMEM((2, _C, _EMB), jnp.float32),
            pltpu.SemaphoreType.DMA,
            pltpu.SemaphoreType.DMA,
            pltpu.SemaphoreType.DMA,
            pltpu.SemaphoreType.DMA,
        ],
    )
    def emb(tok_hbm, table_hbm, out_hbm, idx_v, rows_v, g0, g1, s0, s1):
        wid = lax.axis_index("s") * _NC + lax.axis_index("c")
        base = pl.multiple_of(wid * _PER_W, _PER_W)
        pltpu.sync_copy(tok_hbm.at[pl.ds(base, _PER_W)], idx_v)
        gsems = (g0, g1)
        ssems = (s0, s1)

        def fire_g(ci, buf):
            for j in range(_G):
                pltpu.async_copy(
                    table_hbm.at[idx_v.at[pl.ds(ci * _C + j * _GSZ, _GSZ)]],
                    rows_v.at[buf].at[pl.ds(j * _GSZ, _GSZ)],
                    gsems[buf],
                )

        def wait_g(buf):
            for j in range(_G):
                pltpu.make_async_copy(
                    table_hbm.at[idx_v.at[pl.ds(j * _GSZ, _GSZ)]],
                    rows_v.at[buf].at[pl.ds(j * _GSZ, _GSZ)],
                    gsems[buf],
                ).wait()

        def fire_store(buf, off):
            pltpu.async_copy(rows_v.at[buf], out_hbm.at[pl.ds(off, _C)],
                             ssems[buf])

        def wait_store(buf):
            pltpu.make_async_copy(rows_v.at[buf],
                                  out_hbm.at[pl.ds(base, _C)],
                                  ssems[buf]).wait()

        def scale(buf):
            def row_body(r, c):
                for j in range(_EMB // _LANES):
                    sl = pl.ds(j * _LANES, _LANES)
                    rows_v[buf, r, sl] = rows_v[buf, r, sl] * _SCALE
                return c

            lax.fori_loop(0, _C, row_body, 0, unroll=4)

        fire_g(0, 0)

        def pair_body(k, carry):
            i0 = 2 * k
            off0 = pl.multiple_of(base + i0 * _C, _C)
            off1 = pl.multiple_of(off0 + _C, _C)

            @pl.when(k > 0)
            def _():
                wait_store(1)

            fire_g(i0 + 1, 1)

            wait_g(0)
            scale(0)
            fire_store(0, off0)

            wait_g(1)

            @pl.when(k < _NPAIR - 1)
            def _():
                wait_store(0)
                fire_g(i0 + 2, 0)

            scale(1)
            fire_store(1, off1)
            return carry

        lax.fori_loop(0, _NPAIR, pair_body, 0)
        wait_store(0)
        wait_store(1)

    return emb


_emb = _build()


@jax.jit
def kernel(tokens, table):
    tok = tokens.reshape(_N).astype(jnp.int32)
    out = _emb(tok, table)
    return out.reshape(_B, _L, _EMB)
